# baseline (speedup 1.0000x reference)
import jax
import jax.numpy as jnp
from jax.experimental import pallas as pl
from jax.experimental.pallas import tpu as pltpu

_NBUF = 6
_CPLANES = 4


def _affine_body(ids_ref, w_ref, b_ref, x_hbm, o_hbm, xbuf, obuf, in_sems, out_sems):
    n_chunks = x_hbm.shape[0]
    C = w_ref.shape[1]
    num_sat = w_ref.shape[0]
    H = x_hbm.shape[1] // _CPLANES
    per_sample = C // _CPLANES

    def in_dma(k, slot):
        return pltpu.make_async_copy(x_hbm.at[k], xbuf.at[slot], in_sems.at[slot])

    def out_dma(k, slot):
        return pltpu.make_async_copy(obuf.at[slot], o_hbm.at[k], out_sems.at[slot])

    for k in range(_NBUF):
        in_dma(k, k).start()

    for k in range(n_chunks):
        slot = k % _NBUF
        in_dma(k, slot).wait()
        if k >= _NBUF:
            out_dma(k - _NBUF, slot).wait()

        sid = ids_ref[k // per_sample]
        valid = jnp.logical_and(sid >= 0, sid < num_sat)
        s = jnp.where(valid, sid, 0)
        c0 = (k % per_sample) * _CPLANES
        for j in range(_CPLANES):
            w = jnp.where(valid, w_ref[s, c0 + j], jnp.float32(1.0))
            b = jnp.where(valid, b_ref[s, c0 + j], jnp.float32(0.0))
            obuf[slot, pl.ds(j * H, H), :] = xbuf[slot, pl.ds(j * H, H), :] * w + b
        out_dma(k, slot).start()
        if k + _NBUF < n_chunks:
            in_dma(k + _NBUF, slot).start()

    for k in range(n_chunks - _NBUF, n_chunks):
        out_dma(k, k % _NBUF).wait()


def kernel(x, satellite_ids, weight, bias):
    B, N, C, H, W = x.shape
    S = weight.shape[0]
    n_chunks = B * N * C // _CPLANES
    xr = x.reshape(n_chunks, _CPLANES * H, W)
    ids = satellite_ids.reshape(-1).astype(jnp.int32)
    w2 = weight.reshape(S, C)
    b2 = bias.reshape(S, C)
    grid_spec = pltpu.PrefetchScalarGridSpec(
        num_scalar_prefetch=3,
        grid=(1,),
        in_specs=[pl.BlockSpec(memory_space=pltpu.MemorySpace.HBM)],
        out_specs=pl.BlockSpec(memory_space=pltpu.MemorySpace.HBM),
        scratch_shapes=[
            pltpu.VMEM((_NBUF, _CPLANES * H, W), jnp.float32),
            pltpu.VMEM((_NBUF, _CPLANES * H, W), jnp.float32),
            pltpu.SemaphoreType.DMA((_NBUF,)),
            pltpu.SemaphoreType.DMA((_NBUF,)),
        ],
    )
    out = pl.pallas_call(
        _affine_body,
        grid_spec=grid_spec,
        out_shape=jax.ShapeDtypeStruct((n_chunks, _CPLANES * H, W), x.dtype),
    )(ids, w2, b2, xr)
    return out.reshape(B, N, C, H, W)


# speedup vs baseline: 1.0012x; 1.0012x over previous
import jax
import jax.numpy as jnp
from jax.experimental import pallas as pl
from jax.experimental.pallas import tpu as pltpu

_NBUF = 4
_CPLANES = 4


def _affine_body(ids_ref, w_ref, b_ref, x_hbm, o_hbm, xbuf, obuf, in_sems, out_sems):
    n_chunks = x_hbm.shape[0]
    C = w_ref.shape[1]
    num_sat = w_ref.shape[0]
    H = x_hbm.shape[1] // _CPLANES
    per_sample = C // _CPLANES

    def in_dma(k, slot):
        return pltpu.make_async_copy(x_hbm.at[k], xbuf.at[slot], in_sems.at[slot])

    def out_dma(k, slot):
        return pltpu.make_async_copy(obuf.at[slot], o_hbm.at[k], out_sems.at[slot])

    for k in range(_NBUF):
        in_dma(k, k).start()

    for k in range(n_chunks):
        slot = k % _NBUF
        in_dma(k, slot).wait()
        if k >= _NBUF:
            out_dma(k - _NBUF, slot).wait()

        sid = ids_ref[k // per_sample]
        valid = jnp.logical_and(sid >= 0, sid < num_sat)
        s = jnp.where(valid, sid, 0)
        c0 = (k % per_sample) * _CPLANES
        for j in range(_CPLANES):
            w = jnp.where(valid, w_ref[s, c0 + j], jnp.float32(1.0))
            b = jnp.where(valid, b_ref[s, c0 + j], jnp.float32(0.0))
            obuf[slot, pl.ds(j * H, H), :] = xbuf[slot, pl.ds(j * H, H), :] * w + b
        out_dma(k, slot).start()
        if k + _NBUF < n_chunks:
            in_dma(k + _NBUF, slot).start()

    for k in range(n_chunks - _NBUF, n_chunks):
        out_dma(k, k % _NBUF).wait()


def kernel(x, satellite_ids, weight, bias):
    B, N, C, H, W = x.shape
    S = weight.shape[0]
    n_chunks = B * N * C // _CPLANES
    xr = x.reshape(n_chunks, _CPLANES * H, W)
    ids = satellite_ids.reshape(-1).astype(jnp.int32)
    w2 = weight.reshape(S, C)
    b2 = bias.reshape(S, C)
    grid_spec = pltpu.PrefetchScalarGridSpec(
        num_scalar_prefetch=3,
        grid=(1,),
        in_specs=[pl.BlockSpec(memory_space=pltpu.MemorySpace.HBM)],
        out_specs=pl.BlockSpec(memory_space=pltpu.MemorySpace.HBM),
        scratch_shapes=[
            pltpu.VMEM((_NBUF, _CPLANES * H, W), jnp.float32),
            pltpu.VMEM((_NBUF, _CPLANES * H, W), jnp.float32),
            pltpu.SemaphoreType.DMA((_NBUF,)),
            pltpu.SemaphoreType.DMA((_NBUF,)),
        ],
    )
    out = pl.pallas_call(
        _affine_body,
        grid_spec=grid_spec,
        out_shape=jax.ShapeDtypeStruct((n_chunks, _CPLANES * H, W), x.dtype),
    )(ids, w2, b2, xr)
    return out.reshape(B, N, C, H, W)
